# baseline (speedup 1.0000x reference)
import functools

import jax
import jax.numpy as jnp
from jax import lax
from jax.experimental import pallas as pl
from jax.experimental.pallas import tpu as pltpu
from jax.experimental.pallas import tpu_sc as plsc

_NC = 2
_NS = 16
_L = 16
_NW = _NC * _NS

_N = 16384 * 200
_PER_W = _N // _NW


def _sc_remap(x_flat, table16):
    mesh = plsc.VectorSubcoreMesh(core_axis_name="c", subcore_axis_name="s")

    @functools.partial(
        pl.kernel,
        out_type=jax.ShapeDtypeStruct((_N,), jnp.int32),
        mesh=mesh,
        scratch_types=[
            pltpu.VMEM((_PER_W,), jnp.int32),
            pltpu.VMEM((_L,), jnp.int32),
        ],
        compiler_params=pltpu.CompilerParams(needs_layout_passes=False),
    )
    def k(x_hbm, table_hbm, out_hbm, buf, tbuf):
        wid = lax.axis_index("s") * _NC + lax.axis_index("c")
        base = wid * _PER_W
        pltpu.sync_copy(table_hbm, tbuf)
        pltpu.sync_copy(x_hbm.at[pl.ds(base, _PER_W)], buf)

        tvec = tbuf[...]
        lanes = lax.iota(jnp.int32, 16)
        neg = jnp.int32(-(2**31))

        def lane(j):
            s = jnp.max(jnp.where(lanes == j, tvec, neg))
            return jnp.broadcast_to(s, (_L,))

        t0, t1, t2 = lane(0), lane(1), lane(2)

        def body(i, carry):
            sl = pl.ds(i * _L, _L)
            x = buf[sl]
            buf[sl] = jnp.where(x == 0, t0, jnp.where(x == 1, t1, t2))
            return carry

        lax.fori_loop(0, _PER_W // _L, body, None)
        pltpu.sync_copy(buf, out_hbm.at[pl.ds(base, _PER_W)])

    return k(x_flat, table16)


def kernel(inputs, lookup_table):
    x_flat = inputs.reshape(-1).astype(jnp.int32)
    table16 = jnp.zeros((_L,), jnp.int32).at[:3].set(
        lookup_table.astype(jnp.int32))
    out = _sc_remap(x_flat, table16)
    return out.reshape(inputs.shape).astype(lookup_table.dtype)


# speedup vs baseline: 1.0591x; 1.0591x over previous
import functools

import jax
import jax.numpy as jnp
from jax import lax
from jax.experimental import pallas as pl
from jax.experimental.pallas import tpu as pltpu
from jax.experimental.pallas import tpu_sc as plsc

_NC = 2
_NS = 16
_L = 16
_NW = _NC * _NS

_N = 16384 * 200
_PER_W = _N // _NW
_CH = 25600
_NCH = _PER_W // _CH
_NB = 3
_U = 16


def _sc_remap(x_flat, table16):
    mesh = plsc.VectorSubcoreMesh(core_axis_name="c", subcore_axis_name="s")

    @functools.partial(
        pl.kernel,
        out_type=jax.ShapeDtypeStruct((_N,), jnp.int32),
        mesh=mesh,
        scratch_types=(
            [pltpu.VMEM((_CH,), jnp.int32) for _ in range(_NB)]
            + [pltpu.VMEM((_L,), jnp.int32)]
            + [pltpu.SemaphoreType.DMA for _ in range(2 * _NB)]
        ),
        compiler_params=pltpu.CompilerParams(needs_layout_passes=False),
    )
    def k(x_hbm, table_hbm, out_hbm, b0, b1, b2, tbuf, *sems):
        bufs = (b0, b1, b2)
        isems, osems = sems[:_NB], sems[_NB:]
        wid = lax.axis_index("s") * _NC + lax.axis_index("c")
        base = wid * _PER_W

        def start_in(c):
            return pltpu.async_copy(
                x_hbm.at[pl.ds(base + c * _CH, _CH)], bufs[c % _NB],
                isems[c % _NB])

        ins = {0: start_in(0)}
        outs = {}

        pltpu.sync_copy(table_hbm, tbuf)

        tvec = tbuf[...]
        lanes = lax.iota(jnp.int32, _L)
        neg = jnp.int32(-(2**31))

        def lane(j):
            s = jnp.max(jnp.where(lanes == j, tvec, neg))
            return jnp.broadcast_to(s, (_L,))

        t0, t1, t2 = lane(0), lane(1), lane(2)

        for c in range(_NCH):
            buf = bufs[c % _NB]
            if c + 1 < _NCH:
                if c + 1 >= _NB:
                    outs[c + 1 - _NB].wait()
                ins[c + 1] = start_in(c + 1)
            ins[c].wait()

            def _body(i, carry):
                sl = pl.ds(i * _L, _L)
                x = buf[sl]
                buf[sl] = jnp.where(x == 0, t0, jnp.where(x == 1, t1, t2))
                return carry

            lax.fori_loop(0, _CH // _L, _body, None)

            outs[c] = pltpu.async_copy(
                buf, out_hbm.at[pl.ds(base + c * _CH, _CH)], osems[c % _NB])

        for c in range(max(0, _NCH - _NB), _NCH):
            outs[c].wait()

    return k(x_flat, table16)


def kernel(inputs, lookup_table):
    x_flat = inputs.reshape(-1).astype(jnp.int32)
    table16 = jnp.zeros((_L,), jnp.int32).at[:3].set(
        lookup_table.astype(jnp.int32))
    out = _sc_remap(x_flat, table16)
    return out.reshape(inputs.shape).astype(lookup_table.dtype)


# speedup vs baseline: 1.2792x; 1.2078x over previous
import functools

import jax
import jax.numpy as jnp
from jax import lax
from jax.experimental import pallas as pl
from jax.experimental.pallas import tpu as pltpu
from jax.experimental.pallas import tpu_sc as plsc

_NC = 2
_NS = 16
_L = 16
_NW = _NC * _NS

_N = 16384 * 200
_PER_W = _N // _NW
_CH = 12800
_NCH = _PER_W // _CH
_NB = 3
_U = 16


def _sc_remap(x_flat, table16):
    mesh = plsc.VectorSubcoreMesh(core_axis_name="c", subcore_axis_name="s")

    @functools.partial(
        pl.kernel,
        out_type=jax.ShapeDtypeStruct((_N,), jnp.int32),
        mesh=mesh,
        scratch_types=(
            [pltpu.VMEM((_CH,), jnp.int32) for _ in range(2 * _NB)]
            + [pltpu.VMEM((_L,), jnp.int32)]
            + [pltpu.SemaphoreType.DMA for _ in range(2 * _NB)]
        ),
        compiler_params=pltpu.CompilerParams(needs_layout_passes=False),
    )
    def k(x_hbm, table_hbm, out_hbm, i0, i1, i2, o0, o1, o2, tbuf, *sems):
        ibufs, obufs = (i0, i1, i2), (o0, o1, o2)
        isems, osems = sems[:_NB], sems[_NB:]
        wid = lax.axis_index("s") * _NC + lax.axis_index("c")
        base = wid * _PER_W

        def start_in(c):
            return pltpu.async_copy(
                x_hbm.at[pl.ds(base + c * _CH, _CH)], ibufs[c % _NB],
                isems[c % _NB])

        ins = {0: start_in(0)}
        outs = {}

        pltpu.sync_copy(table_hbm, tbuf)

        tvec = tbuf[...]
        lanes = lax.iota(jnp.int32, _L)
        neg = jnp.int32(-(2**31))

        def lane(j):
            s = jnp.max(jnp.where(lanes == j, tvec, neg))
            return jnp.broadcast_to(s, (_L,))

        t0, t1, t2 = lane(0), lane(1), lane(2)

        for c in range(_NCH):
            ib, ob = ibufs[c % _NB], obufs[c % _NB]
            if c + 1 < _NCH:
                ins[c + 1] = start_in(c + 1)
            ins[c].wait()
            if c >= _NB:
                outs[c - _NB].wait()

            @plsc.parallel_loop(0, _CH, _L, unroll=_U)
            def _(i):
                sl = pl.ds(i, _L)
                x = ib[sl]
                ob[sl] = jnp.where(x == 0, t0, jnp.where(x == 1, t1, t2))

            outs[c] = pltpu.async_copy(
                ob, out_hbm.at[pl.ds(base + c * _CH, _CH)], osems[c % _NB])

        for c in range(max(0, _NCH - _NB), _NCH):
            outs[c].wait()

    return k(x_flat, table16)


def kernel(inputs, lookup_table):
    x_flat = inputs.reshape(-1).astype(jnp.int32)
    table16 = jnp.zeros((_L,), jnp.int32).at[:3].set(
        lookup_table.astype(jnp.int32))
    out = _sc_remap(x_flat, table16)
    return out.reshape(inputs.shape).astype(lookup_table.dtype)


# speedup vs baseline: 2.2402x; 1.7513x over previous
import functools

import jax
import jax.numpy as jnp
from jax import lax
from jax.experimental import pallas as pl
from jax.experimental.pallas import tpu as pltpu
from jax.experimental.pallas import tpu_sc as plsc

_NC = 2
_NS = 16
_L = 16
_NW = _NC * _NS

_ROWS = 16384
_COLS = 200
_RPW = _ROWS // _NW
_CHR = 64
_NCH = _RPW // _CHR
_NB = 3
_U = 2

_OFFS = tuple(range(0, _COLS - _L + 1, _L)) + (_COLS - _L,)


def _sc_remap(x, table16):
    mesh = plsc.VectorSubcoreMesh(core_axis_name="c", subcore_axis_name="s")

    @functools.partial(
        pl.kernel,
        out_type=jax.ShapeDtypeStruct((_ROWS, _COLS), jnp.int32),
        mesh=mesh,
        scratch_types=(
            [pltpu.VMEM((_CHR, _COLS), jnp.int32) for _ in range(2 * _NB)]
            + [pltpu.VMEM((_L,), jnp.int32)]
            + [pltpu.SemaphoreType.DMA for _ in range(2 * _NB)]
        ),
        compiler_params=pltpu.CompilerParams(needs_layout_passes=False),
    )
    def k(x_hbm, table_hbm, out_hbm, i0, i1, i2, o0, o1, o2, tbuf, *sems):
        ibufs, obufs = (i0, i1, i2), (o0, o1, o2)
        isems, osems = sems[:_NB], sems[_NB:]
        wid = lax.axis_index("s") * _NC + lax.axis_index("c")
        rbase = wid * _RPW

        def start_in(c):
            return pltpu.async_copy(
                x_hbm.at[pl.ds(rbase + c * _CHR, _CHR)], ibufs[c % _NB],
                isems[c % _NB])

        ins = {0: start_in(0)}
        outs = {}

        pltpu.sync_copy(table_hbm, tbuf)

        tvec = tbuf[...]
        lanes = lax.iota(jnp.int32, _L)
        neg = jnp.int32(-(2**31))

        def lane(j):
            s = jnp.max(jnp.where(lanes == j, tvec, neg))
            return jnp.broadcast_to(s, (_L,))

        t0, t1, t2 = lane(0), lane(1), lane(2)

        for c in range(_NCH):
            ib, ob = ibufs[c % _NB], obufs[c % _NB]
            if c + 1 < _NCH:
                ins[c + 1] = start_in(c + 1)
            ins[c].wait()
            if c >= _NB:
                outs[c - _NB].wait()

            @plsc.parallel_loop(0, _CHR, 1, unroll=_U)
            def _(r):
                for off in _OFFS:
                    sl = pl.ds(off, _L)
                    xv = ib[r, sl]
                    ob[r, sl] = jnp.where(
                        xv == 0, t0, jnp.where(xv == 1, t1, t2))

            outs[c] = pltpu.async_copy(
                ob, out_hbm.at[pl.ds(rbase + c * _CHR, _CHR)],
                osems[c % _NB])

        for c in range(max(0, _NCH - _NB), _NCH):
            outs[c].wait()

    return k(x, table16)


def kernel(inputs, lookup_table):
    table16 = jnp.zeros((_L,), jnp.int32).at[:3].set(
        lookup_table.astype(jnp.int32))
    out = _sc_remap(inputs.astype(jnp.int32), table16)
    return out.astype(lookup_table.dtype)
